# baseline (speedup 1.0000x reference)
import functools

import jax
import jax.numpy as jnp
from jax import lax
from jax.experimental import pallas as pl
from jax.experimental.pallas import tpu as pltpu
from jax.experimental.pallas import tpu_sc as plsc

B = 4096
E = 64
FEAT = 128
CTX = 32
EPS = 1e-8

NC = 2
NS = 16
NW = NC * NS
BPW = B // NW

BB = 512


@functools.cache
def _make_sc_gather4():
    mesh = plsc.VectorSubcoreMesh(core_axis_name="c", subcore_axis_name="s")

    @functools.partial(
        pl.kernel,
        out_type=tuple(jax.ShapeDtypeStruct((E, B), jnp.float32) for _ in range(4)),
        mesh=mesh,
        scratch_types=[
            pltpu.VMEM((BPW,), jnp.int32),
            pltpu.VMEM((BPW,), jnp.int32),
            pltpu.VMEM((BPW,), jnp.int32),
            pltpu.VMEM((BPW + 16,), jnp.int32),
            pltpu.VMEM((32,), jnp.int32),
            pltpu.VMEM((BPW,), jnp.int32),
            pltpu.VMEM((8, E, 128), jnp.float32),
            pltpu.VMEM((E, BPW), jnp.float32),
            [pltpu.SemaphoreType.DMA] * 8,
        ],
        compiler_params=pltpu.CompilerParams(needs_layout_passes=False),
    )
    def _sc_gather4(users, products, slotid, dcolsf, wstartf,
                    cf_u_t, cf_p_t, nn_u_t, nn_p_t,
                    cf_u_o, cf_p_o, nn_u_o, nn_p_o,
                    uidx, pidx, slot_v, dcol_v, wsv, ue_v, ring, obt, sems):
        wid = lax.axis_index("s") * NC + lax.axis_index("c")
        base = wid * BPW
        sl = pl.ds(base, BPW)
        pltpu.sync_copy(users.at[sl], uidx)
        pltpu.sync_copy(products.at[sl], pidx)
        pltpu.sync_copy(slotid.at[sl], slot_v)
        pltpu.sync_copy(dcolsf.at[sl], dcol_v.at[pl.ds(0, BPW)])
        pltpu.sync_copy(wstartf.at[pl.ds(wid * 32, 32)], wsv)
        lane = lax.iota(jnp.int32, 16)
        for k in range(BPW // 16):
            s16 = pl.ds(k * 16, 16)
            ue_v[s16] = lax.bitwise_and(uidx[s16], 127)

        def user_gather(tbl, out):
            wlo = wsv[pl.ds(0, 16)]
            whi = wsv[pl.ds(16, 16)]
            for vwave in range(16):
                a = wlo[vwave]
                b = whi[0] if vwave == 15 else wlo[vwave + 1]

                @pl.when(a < b)
                def _(a=a, b=b, vwave=vwave, tbl=tbl):
                    dc = dcol_v[pl.ds(8 * vwave, 16)]
                    for j in range(8):
                        pltpu.make_async_copy(
                            tbl.at[:, pl.ds(pl.multiple_of(
                                lax.shift_left(dc[j], 7), 128), 128)],
                            ring.at[j], sems[j]).start()
                    for j in range(8):
                        pltpu.make_async_copy(
                            tbl.at[:, pl.ds(0, 128)],
                            ring.at[j], sems[j]).wait()

                    def sel(il, carry, vwave=vwave):
                        ilv = jnp.full((16,), 0, jnp.int32) + il
                        ev = plsc.load_gather(ue_v, [ilv])
                        sv = lax.bitwise_and(
                            plsc.load_gather(slot_v, [ilv]) - 8 * vwave, 7)
                        for s in range(E // 16):
                            vals = plsc.load_gather(
                                ring, [sv, lane + s * 16, ev])
                            plsc.store_scatter(obt, [lane + s * 16, ilv], vals)
                        return carry

                    lax.fori_loop(a, b, sel, 0)
            pltpu.sync_copy(obt, out.at[:, sl])

        user_gather(cf_u_t, cf_u_o)
        user_gather(nn_u_t, nn_u_o)

        work = ((cf_p_t, pidx, cf_p_o),
                (nn_p_t, pidx, nn_p_o))

        for tbl, idxref, out in work:
            def chunk(ck, carry, tbl=tbl, idxref=idxref):
                v = idxref[pl.ds(ck * 16, 16)]
                col = lax.shift_left(lax.shift_right_logical(v, 7), 7)
                e = lax.bitwise_and(v, 127)

                def fire_quad(q):
                    for j in range(4):
                        slot = (4 * q + j) & 7
                        pltpu.make_async_copy(
                            tbl.at[:, pl.ds(pl.multiple_of(col[4 * q + j], 128), 128)],
                            ring.at[slot], sems[slot]).start()

                def select_quad(q):
                    for j in range(4):
                        slot = (4 * q + j) & 7
                        pltpu.make_async_copy(
                            tbl.at[:, pl.ds(0, 128)],
                            ring.at[slot], sems[slot]).wait()
                        i = ck * 16 + 4 * q + j
                        ej = jnp.full((16,), 0, jnp.int32) + e[4 * q + j]
                        iv = jnp.full((16,), 0, jnp.int32) + i
                        for s in range(E // 16):
                            vals = plsc.load_gather(
                                ring, [jnp.full((16,), slot, jnp.int32),
                                       lane + s * 16, ej])
                            plsc.store_scatter(obt, [lane + s * 16, iv], vals)

                fire_quad(0)
                fire_quad(1)
                select_quad(0)
                fire_quad(2)
                select_quad(1)
                fire_quad(3)
                select_quad(2)
                select_quad(3)
                return carry
            lax.fori_loop(0, BPW // 16, chunk, 0)
            pltpu.sync_copy(obt, out.at[:, sl])

    return _sc_gather4


def _tc_body(cfu, cfp, nnu, nnp, feat, ctx,
             wnnu, wnnp, bnn, wf, bf, wc, bc,
             wcf, wonn, wof, woc, bout, out):
    cfu_ = cfu[...]
    cfp_ = cfp[...]
    dot = jnp.sum(cfu_ * cfp_, axis=0, keepdims=True)
    nu = jnp.maximum(jnp.sqrt(jnp.sum(cfu_ * cfu_, axis=0, keepdims=True)), EPS)
    npn = jnp.maximum(jnp.sqrt(jnp.sum(cfp_ * cfp_, axis=0, keepdims=True)), EPS)
    cf = dot / (nu * npn)
    nn = (jnp.dot(wnnu[...], nnu[...], preferred_element_type=jnp.float32)
          + jnp.dot(wnnp[...], nnp[...], preferred_element_type=jnp.float32)
          + bnn[...])
    nn = jnp.maximum(nn, 0.0)
    fx = jnp.maximum(
        jnp.dot(wf[...], feat[...], preferred_element_type=jnp.float32) + bf[...], 0.0)
    cx = jnp.maximum(
        jnp.dot(wc[...], ctx[...], preferred_element_type=jnp.float32) + bc[...], 0.0)
    y = (cf * wcf[0, 0]
         + jnp.sum(nn * wonn[...], axis=0, keepdims=True)
         + jnp.sum(fx * wof[...], axis=0, keepdims=True)
         + jnp.sum(cx * woc[...], axis=0, keepdims=True)
         + bout[0, 0])
    out[...] = y


def _bt_spec(d):
    return pl.BlockSpec((d, BB), lambda i: (0, i))


def _full_spec(shape):
    return pl.BlockSpec(shape, lambda i: (0, 0))


def kernel(users, products, features, contexts,
           cf_user_emb, cf_product_emb, nn_user_emb, nn_product_emb,
           W_nn, b_nn, W_feat, b_feat, W_ctx, b_ctx, W_out, b_out):
    users = users.astype(jnp.int32)
    products = products.astype(jnp.int32)

    perm = jnp.argsort(users)
    inv = jnp.argsort(perm)
    us = users[perm]
    ps = products[perm]
    ucol = lax.shift_right_logical(us, 7)
    idx = jnp.arange(B, dtype=jnp.int32)
    first = jnp.where(idx % BPW == 0, True, ucol != jnp.roll(ucol, 1))
    css = jnp.cumsum(first.astype(jnp.int32))
    slot = css - css[(idx // BPW) * BPW]
    bounds = jnp.arange(17, dtype=jnp.int32) * 8
    wstart = jax.vmap(
        lambda row: jnp.searchsorted(row, bounds, side="left"))(
            slot.reshape(NW, BPW)).astype(jnp.int32)
    wstart = jnp.pad(wstart, ((0, 0), (0, 15))).reshape(-1)
    dcolsf = jnp.zeros((NW, BPW), jnp.int32).at[
        idx // BPW, slot].set(ucol).reshape(-1)

    cf_u, cf_p, nn_u, nn_p = _make_sc_gather4()(
        us, ps, slot, dcolsf, wstart,
        cf_user_emb.T, cf_product_emb.T, nn_user_emb.T, nn_product_emb.T)

    wnnu = W_nn[:E].T
    wnnp = W_nn[E:].T
    bnn = b_nn[:, None]
    bf = b_feat[:, None]
    bc = b_ctx[:, None]
    wcf = W_out[0:1, 0:1]
    wonn = W_out[1:1 + E // 2, 0][:, None]
    wof = W_out[1 + E // 2:1 + E // 2 + FEAT, 0][:, None]
    woc = W_out[1 + E // 2 + FEAT:, 0][:, None]
    bout = b_out[None, :]

    yt = pl.pallas_call(
        _tc_body,
        grid=(B // BB,),
        in_specs=[
            _bt_spec(E), _bt_spec(E), _bt_spec(E), _bt_spec(E),
            _bt_spec(FEAT), _bt_spec(CTX),
            _full_spec((E // 2, E)), _full_spec((E // 2, E)), _full_spec((E // 2, 1)),
            _full_spec((FEAT, FEAT)), _full_spec((FEAT, 1)),
            _full_spec((CTX, CTX)), _full_spec((CTX, 1)),
            _full_spec((1, 1)), _full_spec((E // 2, 1)),
            _full_spec((FEAT, 1)), _full_spec((CTX, 1)),
            _full_spec((1, 1)),
        ],
        out_specs=pl.BlockSpec((1, BB), lambda i: (0, i)),
        out_shape=jax.ShapeDtypeStruct((1, B), jnp.float32),
    )(cf_u, cf_p, nn_u, nn_p, features[perm].T, contexts[perm].T,
      wnnu, wnnp, bnn, W_feat.T, bf, W_ctx.T, bc,
      wcf, wonn, wof, woc, bout)
    return yt.reshape(B, 1)[inv]


# speedup vs baseline: 1.1513x; 1.1513x over previous
import functools

import jax
import jax.numpy as jnp
from jax import lax
from jax.experimental import pallas as pl
from jax.experimental.pallas import tpu as pltpu
from jax.experimental.pallas import tpu_sc as plsc

B = 4096
E = 64
FEAT = 128
CTX = 32
EPS = 1e-8

NC = 2
NS = 16
NW = NC * NS
BPW = B // NW

BB = 512


@functools.cache
def _make_sc_gather4():
    mesh = plsc.VectorSubcoreMesh(core_axis_name="c", subcore_axis_name="s")

    @functools.partial(
        pl.kernel,
        out_type=tuple(jax.ShapeDtypeStruct((E, B), jnp.float32) for _ in range(4)),
        mesh=mesh,
        scratch_types=[
            pltpu.VMEM((BPW,), jnp.int32),
            pltpu.VMEM((BPW,), jnp.int32),
            pltpu.VMEM((BPW,), jnp.int32),
            pltpu.VMEM((BPW + 16,), jnp.int32),
            pltpu.VMEM((32,), jnp.int32),
            pltpu.VMEM((BPW,), jnp.int32),
            pltpu.VMEM((8, E, 128), jnp.float32),
            pltpu.VMEM((E, BPW), jnp.float32),
            [pltpu.SemaphoreType.DMA] * 8,
        ],
        compiler_params=pltpu.CompilerParams(needs_layout_passes=False),
    )
    def _sc_gather4(users, products, slotid, dcolsf, wstartf,
                    cf_u_t, cf_p_t, nn_u_t, nn_p_t,
                    cf_u_o, cf_p_o, nn_u_o, nn_p_o,
                    uidx, pidx, slot_v, dcol_v, wsv, ue_v, ring, obt, sems):
        wid = lax.axis_index("s") * NC + lax.axis_index("c")
        base = wid * BPW
        sl = pl.ds(base, BPW)
        pltpu.sync_copy(users.at[sl], uidx)
        pltpu.sync_copy(products.at[sl], pidx)
        pltpu.sync_copy(slotid.at[sl], slot_v)
        pltpu.sync_copy(dcolsf.at[sl], dcol_v.at[pl.ds(0, BPW)])
        pltpu.sync_copy(wstartf.at[pl.ds(wid * 32, 32)], wsv)
        lane = lax.iota(jnp.int32, 16)
        for k in range(BPW // 16):
            s16 = pl.ds(k * 16, 16)
            ue_v[s16] = lax.bitwise_and(uidx[s16], 127)

        def user_gather(tbl, out):
            wlo = wsv[pl.ds(0, 16)]
            whi = wsv[pl.ds(16, 16)]
            for vwave in range(16):
                a = wlo[vwave]
                b = whi[0] if vwave == 15 else wlo[vwave + 1]

                @pl.when(a < b)
                def _(a=a, b=b, vwave=vwave, tbl=tbl):
                    dc = dcol_v[pl.ds(8 * vwave, 16)]
                    for j in range(8):
                        pltpu.make_async_copy(
                            tbl.at[:, pl.ds(pl.multiple_of(
                                lax.shift_left(dc[j], 7), 128), 128)],
                            ring.at[j], sems[j]).start()
                    for j in range(8):
                        pltpu.make_async_copy(
                            tbl.at[:, pl.ds(0, 128)],
                            ring.at[j], sems[j]).wait()

                    def sel(il, carry, vwave=vwave):
                        ilv = jnp.full((16,), 0, jnp.int32) + il
                        ev = plsc.load_gather(ue_v, [ilv])
                        sv = lax.bitwise_and(
                            plsc.load_gather(slot_v, [ilv]) - 8 * vwave, 7)
                        for s in range(E // 16):
                            vals = plsc.load_gather(
                                ring, [sv, lane + s * 16, ev])
                            plsc.store_scatter(obt, [lane + s * 16, ilv], vals)
                        return carry

                    lax.fori_loop(a, b, sel, 0)
            pltpu.sync_copy(obt, out.at[:, sl])

        user_gather(cf_u_t, cf_u_o)
        user_gather(nn_u_t, nn_u_o)

        work = ((cf_p_t, pidx, cf_p_o),
                (nn_p_t, pidx, nn_p_o))

        for tbl, idxref, out in work:
            def chunk(ck, carry, tbl=tbl, idxref=idxref):
                v = idxref[pl.ds(ck * 16, 16)]
                col = lax.shift_left(lax.shift_right_logical(v, 7), 7)
                e = lax.bitwise_and(v, 127)

                def fire_quad(q):
                    for j in range(4):
                        slot = (4 * q + j) & 7
                        pltpu.make_async_copy(
                            tbl.at[:, pl.ds(pl.multiple_of(col[4 * q + j], 128), 128)],
                            ring.at[slot], sems[slot]).start()

                def select_quad(q):
                    for j in range(4):
                        slot = (4 * q + j) & 7
                        pltpu.make_async_copy(
                            tbl.at[:, pl.ds(0, 128)],
                            ring.at[slot], sems[slot]).wait()
                        i = ck * 16 + 4 * q + j
                        ej = jnp.full((16,), 0, jnp.int32) + e[4 * q + j]
                        iv = jnp.full((16,), 0, jnp.int32) + i
                        for s in range(E // 16):
                            vals = plsc.load_gather(
                                ring, [jnp.full((16,), slot, jnp.int32),
                                       lane + s * 16, ej])
                            plsc.store_scatter(obt, [lane + s * 16, iv], vals)

                fire_quad(0)
                fire_quad(1)
                select_quad(0)
                fire_quad(2)
                select_quad(1)
                fire_quad(3)
                select_quad(2)
                select_quad(3)
                return carry
            lax.fori_loop(0, BPW // 16, chunk, 0)
            pltpu.sync_copy(obt, out.at[:, sl])

    return _sc_gather4


def _tc_body(cfu, cfp, nnu, nnp, feat, ctx,
             wnnu, wnnp, bnn, wf, bf, wc, bc,
             wcf, wonn, wof, woc, bout, out):
    cfu_ = cfu[...]
    cfp_ = cfp[...]
    dot = jnp.sum(cfu_ * cfp_, axis=0, keepdims=True)
    nu = jnp.maximum(jnp.sqrt(jnp.sum(cfu_ * cfu_, axis=0, keepdims=True)), EPS)
    npn = jnp.maximum(jnp.sqrt(jnp.sum(cfp_ * cfp_, axis=0, keepdims=True)), EPS)
    cf = dot / (nu * npn)
    nn = (jnp.dot(wnnu[...], nnu[...], preferred_element_type=jnp.float32)
          + jnp.dot(wnnp[...], nnp[...], preferred_element_type=jnp.float32)
          + bnn[...])
    nn = jnp.maximum(nn, 0.0)
    fx = jnp.maximum(
        jnp.dot(wf[...], feat[...], preferred_element_type=jnp.float32) + bf[...], 0.0)
    cx = jnp.maximum(
        jnp.dot(wc[...], ctx[...], preferred_element_type=jnp.float32) + bc[...], 0.0)
    y = (cf * wcf[0, 0]
         + jnp.sum(nn * wonn[...], axis=0, keepdims=True)
         + jnp.sum(fx * wof[...], axis=0, keepdims=True)
         + jnp.sum(cx * woc[...], axis=0, keepdims=True)
         + bout[0, 0])
    out[...] = y


def _bt_spec(d):
    return pl.BlockSpec((d, BB), lambda i: (0, i))


def _full_spec(shape):
    return pl.BlockSpec(shape, lambda i: (0, 0))


def kernel(users, products, features, contexts,
           cf_user_emb, cf_product_emb, nn_user_emb, nn_product_emb,
           W_nn, b_nn, W_feat, b_feat, W_ctx, b_ctx, W_out, b_out):
    users = users.astype(jnp.int32)
    products = products.astype(jnp.int32)

    idx = jnp.arange(B, dtype=jnp.int32)
    us, perm, ps = lax.sort((users, idx, products), num_keys=1)
    ucol = lax.shift_right_logical(us, 7)
    first = jnp.where(idx % BPW == 0, True, ucol != jnp.roll(ucol, 1))
    css = jnp.cumsum(first.astype(jnp.int32))
    slot = css - css[(idx // BPW) * BPW]
    slot2d = slot.reshape(NW, BPW)
    starts = jnp.sum(
        slot2d[:, :, None] < jnp.arange(BPW, dtype=jnp.int32)[None, None, :],
        axis=1, dtype=jnp.int32)
    wstart = jnp.concatenate(
        [starts[:, ::8], jnp.full((NW, 1), BPW, jnp.int32),
         jnp.zeros((NW, 15), jnp.int32)], axis=1).reshape(-1)
    dcolsf = jnp.take_along_axis(
        ucol.reshape(NW, BPW), jnp.minimum(starts, BPW - 1), axis=1).reshape(-1)

    cf_u, cf_p, nn_u, nn_p = _make_sc_gather4()(
        us, ps, slot, dcolsf, wstart,
        cf_user_emb.T, cf_product_emb.T, nn_user_emb.T, nn_product_emb.T)

    wnnu = W_nn[:E].T
    wnnp = W_nn[E:].T
    bnn = b_nn[:, None]
    bf = b_feat[:, None]
    bc = b_ctx[:, None]
    wcf = W_out[0:1, 0:1]
    wonn = W_out[1:1 + E // 2, 0][:, None]
    wof = W_out[1 + E // 2:1 + E // 2 + FEAT, 0][:, None]
    woc = W_out[1 + E // 2 + FEAT:, 0][:, None]
    bout = b_out[None, :]

    yt = pl.pallas_call(
        _tc_body,
        grid=(B // BB,),
        in_specs=[
            _bt_spec(E), _bt_spec(E), _bt_spec(E), _bt_spec(E),
            _bt_spec(FEAT), _bt_spec(CTX),
            _full_spec((E // 2, E)), _full_spec((E // 2, E)), _full_spec((E // 2, 1)),
            _full_spec((FEAT, FEAT)), _full_spec((FEAT, 1)),
            _full_spec((CTX, CTX)), _full_spec((CTX, 1)),
            _full_spec((1, 1)), _full_spec((E // 2, 1)),
            _full_spec((FEAT, 1)), _full_spec((CTX, 1)),
            _full_spec((1, 1)),
        ],
        out_specs=pl.BlockSpec((1, BB), lambda i: (0, i)),
        out_shape=jax.ShapeDtypeStruct((1, B), jnp.float32),
    )(cf_u, cf_p, nn_u, nn_p, features[perm].T, contexts[perm].T,
      wnnu, wnnp, bnn, W_feat.T, bf, W_ctx.T, bc,
      wcf, wonn, wof, woc, bout)
    return jnp.zeros((B, 1), jnp.float32).at[perm, 0].set(yt[0])


# speedup vs baseline: 1.2069x; 1.0483x over previous
import functools

import jax
import jax.numpy as jnp
from jax import lax
from jax.experimental import pallas as pl
from jax.experimental.pallas import tpu as pltpu
from jax.experimental.pallas import tpu_sc as plsc

B = 4096
E = 64
FEAT = 128
CTX = 32
EPS = 1e-8

NC = 2
NS = 16
NW = NC * NS
BPW = B // NW

BB = 512


@functools.cache
def _make_sc_gather4():
    mesh = plsc.VectorSubcoreMesh(core_axis_name="c", subcore_axis_name="s")

    @functools.partial(
        pl.kernel,
        out_type=tuple(jax.ShapeDtypeStruct((E, B), jnp.float32) for _ in range(4)),
        mesh=mesh,
        scratch_types=[
            pltpu.VMEM((BPW,), jnp.int32),
            pltpu.VMEM((BPW,), jnp.int32),
            pltpu.VMEM((BPW,), jnp.int32),
            pltpu.VMEM((BPW + 16,), jnp.int32),
            pltpu.VMEM((32,), jnp.int32),
            pltpu.VMEM((BPW,), jnp.int32),
            pltpu.VMEM((8, E, 128), jnp.float32),
            pltpu.VMEM((E, BPW), jnp.float32),
            [pltpu.SemaphoreType.DMA] * 8,
        ],
        compiler_params=pltpu.CompilerParams(needs_layout_passes=False),
    )
    def _sc_gather4(users, products, slotid, dcolsf, wstartf,
                    cf_u_t, cf_p_t, nn_u_t, nn_p_t,
                    cf_u_o, cf_p_o, nn_u_o, nn_p_o,
                    uidx, pidx, slot_v, dcol_v, wsv, ue_v, ring, obt, sems):
        wid = lax.axis_index("s") * NC + lax.axis_index("c")
        base = wid * BPW
        sl = pl.ds(base, BPW)
        pltpu.sync_copy(users.at[sl], uidx)
        pltpu.sync_copy(products.at[sl], pidx)
        pltpu.sync_copy(slotid.at[sl], slot_v)
        pltpu.sync_copy(dcolsf.at[sl], dcol_v.at[pl.ds(0, BPW)])
        pltpu.sync_copy(wstartf.at[pl.ds(wid * 32, 32)], wsv)
        lane = lax.iota(jnp.int32, 16)
        for k in range(BPW // 16):
            s16 = pl.ds(k * 16, 16)
            ue_v[s16] = lax.bitwise_and(uidx[s16], 127)

        def user_gather(tbl, out):
            wlo = wsv[pl.ds(0, 16)]
            whi = wsv[pl.ds(16, 16)]
            for vwave in range(16):
                a = wlo[vwave]
                b = whi[0] if vwave == 15 else wlo[vwave + 1]

                @pl.when(a < b)
                def _(a=a, b=b, vwave=vwave, tbl=tbl):
                    dc = dcol_v[pl.ds(8 * vwave, 16)]
                    for j in range(8):
                        pltpu.make_async_copy(
                            tbl.at[:, pl.ds(pl.multiple_of(
                                lax.shift_left(dc[j], 7), 128), 128)],
                            ring.at[j], sems[j]).start()
                    for j in range(8):
                        pltpu.make_async_copy(
                            tbl.at[:, pl.ds(0, 128)],
                            ring.at[j], sems[j]).wait()

                    def sel(il, carry, vwave=vwave):
                        ilv = jnp.full((16,), 0, jnp.int32) + il
                        ev = plsc.load_gather(ue_v, [ilv])
                        sv = lax.bitwise_and(
                            plsc.load_gather(slot_v, [ilv]) - 8 * vwave, 7)
                        for s in range(E // 16):
                            vals = plsc.load_gather(
                                ring, [sv, lane + s * 16, ev])
                            plsc.store_scatter(obt, [lane + s * 16, ilv], vals)
                        return carry

                    lax.fori_loop(a, b, sel, 0)
            pltpu.sync_copy(obt, out.at[:, sl])

        user_gather(cf_u_t, cf_u_o)
        user_gather(nn_u_t, nn_u_o)

        work = ((cf_p_t, pidx, cf_p_o),
                (nn_p_t, pidx, nn_p_o))

        for tbl, idxref, out in work:
            def chunk(ck, carry, tbl=tbl, idxref=idxref):
                v = idxref[pl.ds(ck * 16, 16)]
                col = lax.shift_left(lax.shift_right_logical(v, 7), 7)
                e = lax.bitwise_and(v, 127)

                def fire_quad(q):
                    for j in range(4):
                        slot = (4 * q + j) & 7
                        pltpu.make_async_copy(
                            tbl.at[:, pl.ds(pl.multiple_of(col[4 * q + j], 128), 128)],
                            ring.at[slot], sems[slot]).start()

                def select_quad(q):
                    for j in range(4):
                        slot = (4 * q + j) & 7
                        pltpu.make_async_copy(
                            tbl.at[:, pl.ds(0, 128)],
                            ring.at[slot], sems[slot]).wait()
                        i = ck * 16 + 4 * q + j
                        ej = jnp.full((16,), 0, jnp.int32) + e[4 * q + j]
                        iv = jnp.full((16,), 0, jnp.int32) + i
                        for s in range(E // 16):
                            vals = plsc.load_gather(
                                ring, [jnp.full((16,), slot, jnp.int32),
                                       lane + s * 16, ej])
                            plsc.store_scatter(obt, [lane + s * 16, iv], vals)

                fire_quad(0)
                fire_quad(1)
                select_quad(0)
                fire_quad(2)
                select_quad(1)
                fire_quad(3)
                select_quad(2)
                select_quad(3)
                return carry
            lax.fori_loop(0, BPW // 16, chunk, 0)
            pltpu.sync_copy(obt, out.at[:, sl])

    return _sc_gather4


def _tc_body(cfu, cfp, nnu, nnp, feat, ctx,
             wnnu, wnnp, bnn, wf, bf, wc, bc,
             wcf, wonn, wof, woc, bout, out):
    cfu_ = cfu[...]
    cfp_ = cfp[...]
    dot = jnp.sum(cfu_ * cfp_, axis=0, keepdims=True)
    nu = jnp.maximum(jnp.sqrt(jnp.sum(cfu_ * cfu_, axis=0, keepdims=True)), EPS)
    npn = jnp.maximum(jnp.sqrt(jnp.sum(cfp_ * cfp_, axis=0, keepdims=True)), EPS)
    cf = dot / (nu * npn)
    nn = (jnp.dot(wnnu[...], nnu[...], preferred_element_type=jnp.float32)
          + jnp.dot(wnnp[...], nnp[...], preferred_element_type=jnp.float32)
          + bnn[...])
    nn = jnp.maximum(nn, 0.0)
    fx = jnp.maximum(
        jnp.dot(feat[...], wf[...], preferred_element_type=jnp.float32) + bf[...], 0.0)
    cx = jnp.maximum(
        jnp.dot(ctx[...], wc[...], preferred_element_type=jnp.float32) + bc[...], 0.0)
    fsum = jnp.sum(fx * wof[...], axis=1, keepdims=True)
    csum = jnp.sum(cx * woc[...], axis=1, keepdims=True)
    y = (cf * wcf[0, 0]
         + jnp.sum(nn * wonn[...], axis=0, keepdims=True)
         + jnp.transpose(fsum + csum)
         + bout[0, 0])
    out[...] = y


def _bt_spec(d):
    return pl.BlockSpec((d, BB), lambda i: (0, i))


def _full_spec(shape):
    return pl.BlockSpec(shape, lambda i: (0, 0))


def kernel(users, products, features, contexts,
           cf_user_emb, cf_product_emb, nn_user_emb, nn_product_emb,
           W_nn, b_nn, W_feat, b_feat, W_ctx, b_ctx, W_out, b_out):
    users = users.astype(jnp.int32)
    products = products.astype(jnp.int32)

    idx = jnp.arange(B, dtype=jnp.int32)
    us, perm, ps = lax.sort((users, idx, products), num_keys=1)
    ucol = lax.shift_right_logical(us, 7)
    first = jnp.where(idx % BPW == 0, True, ucol != jnp.roll(ucol, 1))
    css = jnp.cumsum(first.astype(jnp.int32))
    slot = css - css[(idx // BPW) * BPW]
    slot2d = slot.reshape(NW, BPW)
    starts = jnp.sum(
        slot2d[:, :, None] < jnp.arange(BPW, dtype=jnp.int32)[None, None, :],
        axis=1, dtype=jnp.int32)
    wstart = jnp.concatenate(
        [starts[:, ::8], jnp.full((NW, 1), BPW, jnp.int32),
         jnp.zeros((NW, 15), jnp.int32)], axis=1).reshape(-1)
    dcolsf = jnp.take_along_axis(
        ucol.reshape(NW, BPW), jnp.minimum(starts, BPW - 1), axis=1).reshape(-1)

    cf_u, cf_p, nn_u, nn_p = _make_sc_gather4()(
        us, ps, slot, dcolsf, wstart,
        cf_user_emb.T, cf_product_emb.T, nn_user_emb.T, nn_product_emb.T)

    wnnu = W_nn[:E].T
    wnnp = W_nn[E:].T
    bnn = b_nn[:, None]
    bf = b_feat[None, :]
    bc = b_ctx[None, :]
    wcf = W_out[0:1, 0:1]
    wonn = W_out[1:1 + E // 2, 0][:, None]
    wof = W_out[1 + E // 2:1 + E // 2 + FEAT, 0][None, :]
    woc = W_out[1 + E // 2 + FEAT:, 0][None, :]
    bout = b_out[None, :]

    yt = pl.pallas_call(
        _tc_body,
        grid=(B // BB,),
        in_specs=[
            _bt_spec(E), _bt_spec(E), _bt_spec(E), _bt_spec(E),
            pl.BlockSpec((BB, FEAT), lambda i: (i, 0)),
            pl.BlockSpec((BB, CTX), lambda i: (i, 0)),
            _full_spec((E // 2, E)), _full_spec((E // 2, E)), _full_spec((E // 2, 1)),
            _full_spec((FEAT, FEAT)), _full_spec((1, FEAT)),
            _full_spec((CTX, CTX)), _full_spec((1, CTX)),
            _full_spec((1, 1)), _full_spec((E // 2, 1)),
            _full_spec((1, FEAT)), _full_spec((1, CTX)),
            _full_spec((1, 1)),
        ],
        out_specs=pl.BlockSpec((1, BB), lambda i: (0, i)),
        out_shape=jax.ShapeDtypeStruct((1, B), jnp.float32),
    )(cf_u, cf_p, nn_u, nn_p, features[perm], contexts[perm],
      wnnu, wnnp, bnn, W_feat, bf, W_ctx, bc,
      wcf, wonn, wof, woc, bout)
    inv = lax.sort((perm, idx), num_keys=1)[1]
    return yt[0][inv][:, None]
